# baseline (speedup 1.0000x reference)
import functools
import jax
import jax.numpy as jnp
from jax import lax
from jax.experimental import pallas as pl
from jax.experimental.pallas import tpu as pltpu
from jax.experimental.pallas import tpu_sc as plsc

_C = 1000
_D1 = 26
_B = 16384
_NW = 32
_N = _D1 * _C * _B
_IW = _B // _NW
_OPW = _IW * _D1
_ZBLK = _N // 128


def _zero_block(o_ref):
    o_ref[...] = jnp.zeros(o_ref.shape, jnp.float32)


def _sc_ones(xt_hbm, out_ref, xall, wlist, onesv, wlist2, onesv2, sem, sem2):
    wid = lax.axis_index("s") * 2 + lax.axis_index("c")
    i0 = wid * _IW
    lanes = lax.iota(jnp.int32, 16)
    ones = jnp.ones((16,), jnp.float32)

    for j in range(_D1):
        pltpu.sync_copy(xt_hbm.at[pl.ds(j * _B + i0, _IW)],
                        xall.at[pl.ds(j * _IW, _IW)])

    for j in range(_D1):
        jbase = j * 125
        wl, ov = (wlist, onesv) if j < _D1 // 2 else (wlist2, onesv2)
        jj = j if j < _D1 // 2 else j - _D1 // 2

        def _bj(g, _):
            idx = j * _IW + g * 16
            lidx = jj * _IW + g * 16
            xs = xall[pl.ds(idx, 16)]
            i = i0 + g * 16 + lanes
            w = (((jbase + (xs >> 3)) << 17) + ((i >> 7) << 10)
                 + ((xs & 7) << 7) + (i & 127))
            wl[pl.ds(lidx, 16)] = w
            ov[pl.ds(lidx, 16)] = ones
            return 0

        lax.fori_loop(0, _IW // 16, _bj, 0)

    c1 = pltpu.async_copy(onesv, out_ref.at[wlist], sem)
    c2 = pltpu.async_copy(onesv2, out_ref.at[wlist2], sem2)
    c1.wait()
    c2.wait()


def kernel(x):
    b, c = x.shape
    xt = x.T.astype(jnp.int32).reshape(b * c)

    zeros_flat = pl.pallas_call(
        _zero_block,
        grid=(_N // _ZBLK,),
        out_specs=pl.BlockSpec((_ZBLK,), lambda i: (i,)),
        out_shape=jax.ShapeDtypeStruct((_N,), jnp.float32),
    )()

    scatter = functools.partial(
        pl.kernel,
        mesh=plsc.VectorSubcoreMesh(core_axis_name="c", subcore_axis_name="s"),
        compiler_params=pltpu.CompilerParams(needs_layout_passes=False),
        out_type=(),
        scratch_types=[
            pltpu.VMEM((_OPW,), jnp.int32),
            pltpu.VMEM((_OPW // 2,), jnp.int32),
            pltpu.VMEM((_OPW // 2,), jnp.float32),
            pltpu.VMEM((_OPW // 2,), jnp.int32),
            pltpu.VMEM((_OPW // 2,), jnp.float32),
            pltpu.SemaphoreType.DMA,
            pltpu.SemaphoreType.DMA,
        ],
    )(_sc_ones)

    def run(xt_arr, zf):
        r = jax.new_ref(zf)
        scatter(xt_arr, r)
        return r[...]

    out = run(xt, zeros_flat)
    out5 = out.reshape(_D1, _C // 8, _B // 128, 8, 128)
    return out5.transpose(2, 4, 0, 1, 3).reshape(_B, _D1, _C)
